# initial kernel scaffold (unmeasured)
import jax
import jax.numpy as jnp
from jax import lax
from jax.experimental import pallas as pl
from jax.experimental.pallas import tpu as pltpu

N_DEV = 4


def kernel(x, w_mat, scale_x, scale_w):
    m_per, k = x.shape
    _, n = w_mat.shape

    def body(x_ref, w_ref, sx_ref, sw_ref, out_ref, comm_ref, send_sems, recv_sems):
        my_pos = lax.axis_index("i")
        left = lax.rem(my_pos + N_DEV - 1, N_DEV)
        right = lax.rem(my_pos + 1, N_DEV)

        barrier_sem = pltpu.get_barrier_semaphore()
        for nbr in (left, right):
            pl.semaphore_signal(
                barrier_sem, inc=1,
                device_id=(nbr,), device_id_type=pl.DeviceIdType.MESH,
            )
        pl.semaphore_wait(barrier_sem, 2)

        scale = sx_ref[0] * sw_ref[0]

        def gemm_store(chunk, origin):
            acc = jnp.dot(chunk, w_ref[...], preferred_element_type=jnp.int32)
            y = acc.astype(jnp.float32) * scale
            out_ref[pl.ds(origin * m_per, m_per), :] = jnp.maximum(y, 0.0)

        rdma = pltpu.make_async_remote_copy(
            src_ref=x_ref,
            dst_ref=comm_ref.at[0],
            send_sem=send_sems.at[0],
            recv_sem=recv_sems.at[0],
            device_id=(right,),
            device_id_type=pl.DeviceIdType.MESH,
        )
        rdma.start()
        gemm_store(x_ref[...], my_pos)
        rdma.wait()

        for h in range(1, N_DEV - 1):
            rdma = pltpu.make_async_remote_copy(
                src_ref=comm_ref.at[h - 1],
                dst_ref=comm_ref.at[h],
                send_sem=send_sems.at[h],
                recv_sem=recv_sems.at[h],
                device_id=(right,),
                device_id_type=pl.DeviceIdType.MESH,
            )
            rdma.start()
            gemm_store(comm_ref[h - 1], lax.rem(my_pos - h + N_DEV, N_DEV))
            rdma.wait()

        gemm_store(
            comm_ref[N_DEV - 2],
            lax.rem(my_pos - (N_DEV - 1) + N_DEV, N_DEV),
        )

    return pl.pallas_call(
        body,
        out_shape=jax.ShapeDtypeStruct((N_DEV * m_per, n), jnp.float32),
        in_specs=[
            pl.BlockSpec(memory_space=pltpu.VMEM),
            pl.BlockSpec(memory_space=pltpu.VMEM),
            pl.BlockSpec(memory_space=pltpu.SMEM),
            pl.BlockSpec(memory_space=pltpu.SMEM),
        ],
        out_specs=pl.BlockSpec(memory_space=pltpu.VMEM),
        scratch_shapes=[
            pltpu.VMEM((N_DEV - 1, m_per, k), x.dtype),
            pltpu.SemaphoreType.DMA((N_DEV - 1,)),
            pltpu.SemaphoreType.DMA((N_DEV - 1,)),
        ],
        compiler_params=pltpu.CompilerParams(collective_id=0),
    )(x, w_mat, scale_x, scale_w)


# baseline (device time: 192393 ns/iter reference)
import jax
import jax.numpy as jnp
from jax import lax
from jax.experimental import pallas as pl
from jax.experimental.pallas import tpu as pltpu

N_DEV = 4


def kernel(x, w_mat, scale_x, scale_w):
    m_per, k = x.shape
    _, n = w_mat.shape

    def body(x_ref, w_ref, sx_ref, sw_ref, out_ref,
             comm_ref, stage_ref, send_sems, recv_sems, copy_sems):
        my_pos = lax.axis_index("i")
        left = lax.rem(my_pos + N_DEV - 1, N_DEV)
        right = lax.rem(my_pos + 1, N_DEV)

        barrier_sem = pltpu.get_barrier_semaphore()
        for nbr in (left, right):
            pl.semaphore_signal(
                barrier_sem, inc=1,
                device_id=(nbr,), device_id_type=pl.DeviceIdType.MESH,
            )
        pl.semaphore_wait(barrier_sem, 2)

        scale = sx_ref[0] * sw_ref[0]
        pending = [None, None]

        def gemm_store(chunk, origin, h):
            slot = h % 2
            if pending[slot] is not None:
                pending[slot].wait()
            acc = jnp.dot(chunk, w_ref[...], preferred_element_type=jnp.int32)
            stage_ref[slot] = jnp.maximum(acc.astype(jnp.float32) * scale, 0.0)
            cp = pltpu.make_async_copy(
                stage_ref.at[slot],
                out_ref.at[pl.ds(origin * m_per, m_per), :],
                copy_sems.at[slot],
            )
            cp.start()
            pending[slot] = cp

        rdma = pltpu.make_async_remote_copy(
            src_ref=x_ref,
            dst_ref=comm_ref.at[0],
            send_sem=send_sems.at[0],
            recv_sem=recv_sems.at[0],
            device_id=(right,),
            device_id_type=pl.DeviceIdType.MESH,
        )
        rdma.start()
        gemm_store(x_ref[...], my_pos, 0)
        rdma.wait()

        for h in range(1, N_DEV - 1):
            rdma = pltpu.make_async_remote_copy(
                src_ref=comm_ref.at[h - 1],
                dst_ref=comm_ref.at[h],
                send_sem=send_sems.at[h],
                recv_sem=recv_sems.at[h],
                device_id=(right,),
                device_id_type=pl.DeviceIdType.MESH,
            )
            rdma.start()
            gemm_store(comm_ref[h - 1], lax.rem(my_pos - h + N_DEV, N_DEV), h)
            rdma.wait()

        gemm_store(
            comm_ref[N_DEV - 2],
            lax.rem(my_pos + 1, N_DEV),
            N_DEV - 1,
        )
        for cp in pending:
            cp.wait()

    return pl.pallas_call(
        body,
        out_shape=jax.ShapeDtypeStruct((N_DEV * m_per, n), jnp.float32),
        in_specs=[
            pl.BlockSpec(memory_space=pltpu.VMEM),
            pl.BlockSpec(memory_space=pltpu.VMEM),
            pl.BlockSpec(memory_space=pltpu.SMEM),
            pl.BlockSpec(memory_space=pltpu.SMEM),
        ],
        out_specs=pl.BlockSpec(memory_space=pl.ANY),
        scratch_shapes=[
            pltpu.VMEM((N_DEV - 1, m_per, k), x.dtype),
            pltpu.VMEM((2, m_per, n), jnp.float32),
            pltpu.SemaphoreType.DMA((N_DEV - 1,)),
            pltpu.SemaphoreType.DMA((N_DEV - 1,)),
            pltpu.SemaphoreType.DMA((2,)),
        ],
        compiler_params=pltpu.CompilerParams(
            collective_id=0,
            vmem_limit_bytes=100 * 1024 * 1024,
        ),
    )(x, w_mat, scale_x, scale_w)


# device time: 125608 ns/iter; 1.5317x vs baseline; 1.5317x over previous
import jax
import jax.numpy as jnp
from jax import lax
from jax.experimental import pallas as pl
from jax.experimental.pallas import tpu as pltpu

N_DEV = 4
N_HOP = N_DEV - 1


def kernel(x, w_mat, scale_x, scale_w):
    m_per, k = x.shape
    half = m_per // 2
    _, n = w_mat.shape

    def body(x_ref, w_ref, sx_ref, sw_ref, out_ref,
             cw_ref, ccw_ref, stage_ref,
             cw_send_sems, cw_recv_sems, ccw_send_sems, ccw_recv_sems,
             copy_sems):
        my_pos = lax.axis_index("i")
        left = lax.rem(my_pos + N_DEV - 1, N_DEV)
        right = lax.rem(my_pos + 1, N_DEV)

        barrier_sem = pltpu.get_barrier_semaphore()
        for nbr in (left, right):
            pl.semaphore_signal(
                barrier_sem, inc=1,
                device_id=(nbr,), device_id_type=pl.DeviceIdType.MESH,
            )
        pl.semaphore_wait(barrier_sem, 2)

        scale = sx_ref[0] * sw_ref[0]
        pending = [None, None]
        unit = [0]

        def gemm_store(chunk, row_start):
            slot = unit[0] % 2
            unit[0] += 1
            if pending[slot] is not None:
                pending[slot].wait()
            acc = jnp.dot(chunk, w_ref[...], preferred_element_type=jnp.int32)
            stage_ref[slot] = jnp.maximum(acc.astype(jnp.float32) * scale, 0.0)
            cp = pltpu.make_async_copy(
                stage_ref.at[slot],
                out_ref.at[pl.ds(row_start, half), :],
                copy_sems.at[slot],
            )
            cp.start()
            pending[slot] = cp

        def make_hop(h):
            cw_src = x_ref.at[pl.ds(0, half), :] if h == 0 else cw_ref.at[h - 1]
            ccw_src = (
                x_ref.at[pl.ds(half, half), :] if h == 0 else ccw_ref.at[h - 1]
            )
            cw = pltpu.make_async_remote_copy(
                src_ref=cw_src,
                dst_ref=cw_ref.at[h],
                send_sem=cw_send_sems.at[h],
                recv_sem=cw_recv_sems.at[h],
                device_id=(right,),
                device_id_type=pl.DeviceIdType.MESH,
            )
            ccw = pltpu.make_async_remote_copy(
                src_ref=ccw_src,
                dst_ref=ccw_ref.at[h],
                send_sem=ccw_send_sems.at[h],
                recv_sem=ccw_recv_sems.at[h],
                device_id=(left,),
                device_id_type=pl.DeviceIdType.MESH,
            )
            return cw, ccw

        rdmas = []

        cw, ccw = make_hop(0)
        cw.start()
        ccw.start()
        rdmas.append((cw, ccw))
        gemm_store(x_ref[pl.ds(0, half), :], my_pos * m_per)
        gemm_store(x_ref[pl.ds(half, half), :], my_pos * m_per + half)

        for h in range(N_HOP):
            cw, ccw = rdmas[h]
            cw.wait_recv()
            ccw.wait_recv()
            if h + 1 < N_HOP:
                nxt = make_hop(h + 1)
                nxt[0].start()
                nxt[1].start()
                rdmas.append(nxt)
            cw_origin = lax.rem(my_pos - (h + 1) + N_DEV, N_DEV)
            ccw_origin = lax.rem(my_pos + (h + 1), N_DEV)
            gemm_store(cw_ref[h], cw_origin * m_per)
            gemm_store(ccw_ref[h], ccw_origin * m_per + half)

        for cw, ccw in rdmas:
            cw.wait_send()
            ccw.wait_send()
        for cp in pending:
            cp.wait()

    return pl.pallas_call(
        body,
        out_shape=jax.ShapeDtypeStruct((N_DEV * m_per, n), jnp.float32),
        in_specs=[
            pl.BlockSpec(memory_space=pltpu.VMEM),
            pl.BlockSpec(memory_space=pltpu.VMEM),
            pl.BlockSpec(memory_space=pltpu.SMEM),
            pl.BlockSpec(memory_space=pltpu.SMEM),
        ],
        out_specs=pl.BlockSpec(memory_space=pl.ANY),
        scratch_shapes=[
            pltpu.VMEM((N_HOP, half, k), x.dtype),
            pltpu.VMEM((N_HOP, half, k), x.dtype),
            pltpu.VMEM((2, half, n), jnp.float32),
            pltpu.SemaphoreType.DMA((N_HOP,)),
            pltpu.SemaphoreType.DMA((N_HOP,)),
            pltpu.SemaphoreType.DMA((N_HOP,)),
            pltpu.SemaphoreType.DMA((N_HOP,)),
            pltpu.SemaphoreType.DMA((2,)),
        ],
        compiler_params=pltpu.CompilerParams(
            collective_id=0,
            vmem_limit_bytes=100 * 1024 * 1024,
        ),
    )(x, w_mat, scale_x, scale_w)


# device time: 110645 ns/iter; 1.7388x vs baseline; 1.1352x over previous
import jax
import jax.numpy as jnp
from jax import lax
from jax.experimental import pallas as pl
from jax.experimental.pallas import tpu as pltpu

N_DEV = 4
N_SUB = 4
N_STAGE = 4


def kernel(x, w_mat, scale_x, scale_w):
    m_per, k = x.shape
    sub = m_per // N_SUB
    half = m_per // 2
    _, n = w_mat.shape

    def body(x_ref, w_ref, sx_ref, sw_ref, out_ref,
             l_ref, r_ref, d_ref, stage_ref,
             own_r_send, own_l_send, fwd_r_send, fwd_l_send,
             nb_l_recv, nb_r_recv, diag_l_recv, diag_r_recv,
             copy_sems):
        my_pos = lax.axis_index("i")
        left = lax.rem(my_pos + N_DEV - 1, N_DEV)
        right = lax.rem(my_pos + 1, N_DEV)

        barrier_sem = pltpu.get_barrier_semaphore()
        for nbr in (left, right):
            pl.semaphore_signal(
                barrier_sem, inc=1,
                device_id=(nbr,), device_id_type=pl.DeviceIdType.MESH,
            )
        pl.semaphore_wait(barrier_sem, 2)

        scale = sx_ref[0] * sw_ref[0]
        pending = [None] * N_STAGE
        unit = [0]
        rdmas = []

        def gemm_store(chunk, row_start):
            slot = unit[0] % N_STAGE
            unit[0] += 1
            if pending[slot] is not None:
                pending[slot].wait()
            acc = jnp.dot(chunk, w_ref[...], preferred_element_type=jnp.int32)
            stage_ref[slot] = jnp.maximum(acc.astype(jnp.float32) * scale, 0.0)
            cp = pltpu.make_async_copy(
                stage_ref.at[slot],
                out_ref.at[pl.ds(row_start, sub), :],
                copy_sems.at[slot],
            )
            cp.start()
            pending[slot] = cp

        def rdma_start(src, dst, ssem, rsem, tgt):
            r = pltpu.make_async_remote_copy(
                src_ref=src, dst_ref=dst, send_sem=ssem, recv_sem=rsem,
                device_id=(tgt,), device_id_type=pl.DeviceIdType.MESH,
            )
            r.start()
            rdmas.append(r)
            return r

        def sl(ref, s):
            return ref.at[pl.ds(s * sub, sub), :]

        def slv(ref, s):
            return ref[pl.ds(s * sub, sub), :]

        nb_l = [None] * N_SUB
        nb_r = [None] * N_SUB
        for s in range(N_SUB):
            rdma_start(sl(x_ref, s), sl(l_ref, s),
                       own_r_send.at[s], nb_l_recv.at[s], right)
            rdma_start(sl(x_ref, s), sl(r_ref, s),
                       own_l_send.at[s], nb_r_recv.at[s], left)
        for s in range(N_SUB):
            nb_l[s] = pltpu.make_async_remote_copy(
                src_ref=sl(x_ref, s), dst_ref=sl(l_ref, s),
                send_sem=own_r_send.at[s], recv_sem=nb_l_recv.at[s],
                device_id=(right,), device_id_type=pl.DeviceIdType.MESH,
            )
            nb_r[s] = pltpu.make_async_remote_copy(
                src_ref=sl(x_ref, s), dst_ref=sl(r_ref, s),
                send_sem=own_l_send.at[s], recv_sem=nb_r_recv.at[s],
                device_id=(left,), device_id_type=pl.DeviceIdType.MESH,
            )

        for s in range(N_SUB):
            gemm_store(slv(x_ref, s), my_pos * m_per + s * sub)

        for s in range(N_SUB):
            nb_l[s].wait_recv()
            if s < 2:
                rdma_start(sl(l_ref, s), sl(d_ref, s),
                           fwd_r_send.at[s], diag_l_recv.at[s], right)
            nb_r[s].wait_recv()
            if s >= 2:
                rdma_start(sl(r_ref, s), sl(d_ref, s),
                           fwd_l_send.at[s - 2], diag_r_recv.at[s - 2], left)
            origin_l = left
            origin_r = right
            gemm_store(slv(l_ref, s), origin_l * m_per + s * sub)
            gemm_store(slv(r_ref, s), origin_r * m_per + s * sub)

        diag = lax.rem(my_pos + 2, N_DEV)
        for s in range(2):
            d_top = pltpu.make_async_remote_copy(
                src_ref=sl(l_ref, s), dst_ref=sl(d_ref, s),
                send_sem=fwd_r_send.at[s], recv_sem=diag_l_recv.at[s],
                device_id=(right,), device_id_type=pl.DeviceIdType.MESH,
            )
            d_top.wait_recv()
            gemm_store(slv(d_ref, s), diag * m_per + s * sub)
            d_bot = pltpu.make_async_remote_copy(
                src_ref=sl(r_ref, 2 + s), dst_ref=sl(d_ref, 2 + s),
                send_sem=fwd_l_send.at[s], recv_sem=diag_r_recv.at[s],
                device_id=(left,), device_id_type=pl.DeviceIdType.MESH,
            )
            d_bot.wait_recv()
            gemm_store(slv(d_ref, 2 + s), diag * m_per + (2 + s) * sub)

        for r in rdmas:
            r.wait_send()
        for cp in pending:
            if cp is not None:
                cp.wait()

    return pl.pallas_call(
        body,
        out_shape=jax.ShapeDtypeStruct((N_DEV * m_per, n), jnp.float32),
        in_specs=[
            pl.BlockSpec(memory_space=pltpu.VMEM),
            pl.BlockSpec(memory_space=pltpu.VMEM),
            pl.BlockSpec(memory_space=pltpu.SMEM),
            pl.BlockSpec(memory_space=pltpu.SMEM),
        ],
        out_specs=pl.BlockSpec(memory_space=pl.ANY),
        scratch_shapes=[
            pltpu.VMEM((m_per, k), x.dtype),
            pltpu.VMEM((m_per, k), x.dtype),
            pltpu.VMEM((m_per, k), x.dtype),
            pltpu.VMEM((N_STAGE, sub, n), jnp.float32),
            pltpu.SemaphoreType.DMA((N_SUB,)),
            pltpu.SemaphoreType.DMA((N_SUB,)),
            pltpu.SemaphoreType.DMA((2,)),
            pltpu.SemaphoreType.DMA((2,)),
            pltpu.SemaphoreType.DMA((N_SUB,)),
            pltpu.SemaphoreType.DMA((N_SUB,)),
            pltpu.SemaphoreType.DMA((2,)),
            pltpu.SemaphoreType.DMA((2,)),
            pltpu.SemaphoreType.DMA((N_STAGE,)),
        ],
        compiler_params=pltpu.CompilerParams(
            collective_id=0,
            vmem_limit_bytes=100 * 1024 * 1024,
        ),
    )(x, w_mat, scale_x, scale_w)
